# baseline (speedup 1.0000x reference)
import functools
import math

import jax
import jax.numpy as jnp
import numpy as np
from jax import lax
from jax.experimental import pallas as pl
from jax.experimental.pallas import tpu as pltpu
from jax.experimental.pallas import tpu_sc as plsc

_IN_CH = 128
_HID = 16
_LAT = 512
_NUM_LAYERS = 10
_GNN_LAYERS = 8
_DIM_ADD = (_LAT - _HID) // _GNN_LAYERS
_CH = [_HID] + [_HID + i * _DIM_ADD for i in range(1, _NUM_LAYERS)] + [_LAT]
_B = 10
_NG = 1000
_N = _B * _NG
_E = 160000
_RATIO = 0.9
_GIN_EPS = 3.0
_BN_EPS = 1e-4

_NSC = 2
_NTILES = 16
_KCH = 128
_BM = 256


def _rup(v, m):
    return ((v + m - 1) // m) * m


@functools.lru_cache(maxsize=None)
def _make_scatter(n_pad, csc, e_pad):
    ew = e_pad // _NTILES
    nchunks = ew // _KCH
    rows_pt = n_pad // _NTILES
    assert ew % _KCH == 0 and n_pad % (16 * _NTILES) == 0
    mesh = plsc.VectorSubcoreMesh(core_axis_name="c", subcore_axis_name="s")

    @functools.partial(
        pl.kernel,
        mesh=mesh,
        compiler_params=pltpu.CompilerParams(use_tc_tiling_on_sc=False),
        out_type=jax.ShapeDtypeStruct((_NSC * n_pad, csc), jnp.float32),
        scratch_types=[
            pltpu.VMEM((_KCH,), jnp.int32),
            pltpu.VMEM((_KCH,), jnp.int32),
            pltpu.VMEM((_KCH, csc), jnp.float32),
            pltpu.VMEM_SHARED((n_pad, csc), jnp.float32),
            pltpu.SemaphoreType.DMA,
        ],
    )
    def k(src_ref, dst_ref, x_ref, out_ref, sidx, didx, buf, acc, sem):
        sc = lax.axis_index("c")
        s = lax.axis_index("s")
        zero = jnp.zeros((16,), jnp.float32)
        for j in range(16):
            for cb in range(csc // 16):
                buf[j, cb * 16:(cb + 1) * 16] = zero
        r0 = s * rows_pt

        def zinit(i, carry):
            pltpu.sync_copy(buf.at[pl.ds(0, 16)],
                            acc.at[pl.ds(r0 + i * 16, 16)])
            return carry

        lax.fori_loop(0, rows_pt // 16, zinit, 0)
        plsc.subcore_barrier()

        ebase = s * ew
        sbase = sc * e_pad + ebase

        def step(i, carry):
            pltpu.sync_copy(src_ref.at[pl.ds(sbase + i * _KCH, _KCH)], sidx)
            pltpu.sync_copy(dst_ref.at[pl.ds(ebase + i * _KCH, _KCH)], didx)
            pltpu.async_copy(x_ref.at[sidx], buf, sem).wait()
            pltpu.sync_copy(buf, acc.at[didx], add=True)
            return carry

        lax.fori_loop(0, nchunks, step, 0)
        plsc.subcore_barrier()
        pltpu.sync_copy(acc.at[pl.ds(r0, rows_pt)],
                        out_ref.at[pl.ds(sc * n_pad + r0, rows_pt)])

    return k


def _scatter_sum(x, src_m, dst_m, n, c):
    csc = max(16, _rup(c, 32) // 2)
    cpad = 2 * csc
    n_pad = _rup(n + 1, 16 * _NTILES)
    e = src_m.shape[0]
    e_pad = _rup(e, _NTILES * _KCH)
    xs = jnp.pad(x, ((0, n_pad - n), (0, cpad - c)))
    xs = xs.reshape(n_pad, 2, csc).transpose(1, 0, 2).reshape(2 * n_pad, csc)
    src_p = jnp.pad(src_m, (0, e_pad - e), constant_values=n)
    dst_p = jnp.pad(dst_m, (0, e_pad - e), constant_values=n)
    srco = jnp.concatenate([src_p, src_p + n_pad])
    out2 = _make_scatter(n_pad, csc, e_pad)(srco, dst_p, xs)
    out = out2.reshape(2, n_pad, csc).transpose(1, 0, 2).reshape(n_pad, cpad)
    return out[:n, :c]


def _dot(a, b):
    return jnp.dot(a, b)


@functools.lru_cache(maxsize=None)
def _make_linear(m_pad, kp, np_):
    def body(x_ref, w_ref, b_ref, o_ref):
        o_ref[...] = _dot(x_ref[...], w_ref[...]) + b_ref[...]

    return pl.pallas_call(
        body,
        grid=(m_pad // _BM,),
        in_specs=[
            pl.BlockSpec((_BM, kp), lambda i: (i, 0)),
            pl.BlockSpec((kp, np_), lambda i: (0, 0)),
            pl.BlockSpec((1, np_), lambda i: (0, 0)),
        ],
        out_specs=pl.BlockSpec((_BM, np_), lambda i: (i, 0)),
        out_shape=jax.ShapeDtypeStruct((m_pad, np_), jnp.float32),
    )


def _linear(x, w, b):
    (n, ci), co = x.shape, w.shape[1]
    m_pad, kp, np_ = _rup(n, _BM), _rup(ci, 128), _rup(co, 128)
    xp = jnp.pad(x, ((0, m_pad - n), (0, kp - ci)))
    wp = jnp.pad(w, ((0, kp - ci), (0, np_ - co)))
    bp = jnp.pad(b.reshape(1, -1), ((0, 0), (0, np_ - co)))
    return _make_linear(m_pad, kp, np_)(xp, wp, bp)[:n, :co]


@functools.lru_cache(maxsize=None)
def _make_gin_linear(m_pad, kp, np_):
    def body(x_ref, a_ref, w_ref, b_ref, o_ref):
        h = (1.0 + _GIN_EPS) * x_ref[...] + a_ref[...]
        o_ref[...] = _dot(h, w_ref[...]) + b_ref[...]

    return pl.pallas_call(
        body,
        grid=(m_pad // _BM,),
        in_specs=[
            pl.BlockSpec((_BM, kp), lambda i: (i, 0)),
            pl.BlockSpec((_BM, kp), lambda i: (i, 0)),
            pl.BlockSpec((kp, np_), lambda i: (0, 0)),
            pl.BlockSpec((1, np_), lambda i: (0, 0)),
        ],
        out_specs=pl.BlockSpec((_BM, np_), lambda i: (i, 0)),
        out_shape=jax.ShapeDtypeStruct((m_pad, np_), jnp.float32),
    )


def _gin_linear(x, agg, w, b):
    (n, ci), co = x.shape, w.shape[1]
    m_pad, kp, np_ = _rup(n, _BM), _rup(ci, 128), _rup(co, 128)
    xp = jnp.pad(x, ((0, m_pad - n), (0, kp - ci)))
    ap = jnp.pad(agg, ((0, m_pad - n), (0, kp - ci)))
    wp = jnp.pad(w, ((0, kp - ci), (0, np_ - co)))
    bp = jnp.pad(b.reshape(1, -1), ((0, 0), (0, np_ - co)))
    return _make_gin_linear(m_pad, kp, np_)(xp, ap, wp, bp)[:n, :co]


@functools.lru_cache(maxsize=None)
def _make_score(m_pad, kp):
    def body(a_ref, x_ref, wa_ref, wb_ref, b_ref, o_ref):
        pre = _dot(a_ref[...], wa_ref[...]) + _dot(x_ref[...], wb_ref[...])
        o_ref[...] = jnp.tanh(pre + b_ref[...])

    return pl.pallas_call(
        body,
        grid=(m_pad // _BM,),
        in_specs=[
            pl.BlockSpec((_BM, kp), lambda i: (i, 0)),
            pl.BlockSpec((_BM, kp), lambda i: (i, 0)),
            pl.BlockSpec((kp, 128), lambda i: (0, 0)),
            pl.BlockSpec((kp, 128), lambda i: (0, 0)),
            pl.BlockSpec((1, 128), lambda i: (0, 0)),
        ],
        out_specs=pl.BlockSpec((_BM, 128), lambda i: (i, 0)),
        out_shape=jax.ShapeDtypeStruct((m_pad, 128), jnp.float32),
    )


def _score(aggs, x, w_rel, b_rel, w_root):
    n, c = x.shape
    m_pad, kp = _rup(n, _BM), _rup(c, 128)
    ap = jnp.pad(aggs, ((0, m_pad - n), (0, kp - c)))
    xp = jnp.pad(x, ((0, m_pad - n), (0, kp - c)))
    wa = jnp.pad(w_rel, ((0, kp - c), (0, 127)))
    wb = jnp.pad(w_root, ((0, kp - c), (0, 127)))
    bp = jnp.pad(b_rel.reshape(1, 1), ((0, 0), (0, 127)))
    return _make_score(m_pad, kp)(ap, xp, wa, wb, bp)[:n, 0]


@functools.lru_cache(maxsize=None)
def _make_bn_scale(m_pad, cp, n_real):
    def body(z_ref, s_ref, g_ref, b_ref, o_ref):
        zz = z_ref[...] * s_ref[:, 0:1]
        mean = jnp.sum(zz, axis=0, keepdims=True) / n_real
        ex2 = jnp.sum(zz * zz, axis=0, keepdims=True) / n_real
        var = ex2 - mean * mean
        inv = lax.rsqrt(var + _BN_EPS)
        o_ref[...] = (zz - mean) * inv * g_ref[...] + b_ref[...]

    return pl.pallas_call(
        body,
        grid=(1,),
        in_specs=[
            pl.BlockSpec((m_pad, cp), lambda i: (0, 0)),
            pl.BlockSpec((m_pad, 128), lambda i: (0, 0)),
            pl.BlockSpec((1, cp), lambda i: (0, 0)),
            pl.BlockSpec((1, cp), lambda i: (0, 0)),
        ],
        out_specs=pl.BlockSpec((m_pad, cp), lambda i: (0, 0)),
        out_shape=jax.ShapeDtypeStruct((m_pad, cp), jnp.float32),
    )


def _bn_scale(z, s, g, b):
    n, c = z.shape
    m_pad, cp = _rup(n, 8), _rup(c, 128)
    zp = jnp.pad(z, ((0, m_pad - n), (0, cp - c)))
    sp = jnp.pad(s.reshape(-1, 1), ((0, m_pad - n), (0, 127)))
    gp = jnp.pad(g.reshape(1, -1), ((0, 0), (0, cp - c)),
                 constant_values=1.0)
    bp = jnp.pad(b.reshape(1, -1), ((0, 0), (0, cp - c)))
    return _make_bn_scale(m_pad, cp, n)(zp, sp, gp, bp)[:n, :c]


def _topk_perm(score, b, ng, kk):
    _, idx = lax.top_k(score.reshape(b, ng), kk)
    return (idx + (jnp.arange(b) * ng)[:, None]).reshape(-1)


def _forward(x, params, src, dst):
    valid = jnp.ones((_E,), bool)
    x = _linear(x, params['W0'], params['b0'])
    ng = _NG
    for i in range(_NUM_LAYERS):
        n = x.shape[0]
        src_m = jnp.where(valid, src, n)
        dst_m = jnp.where(valid, dst, n)
        ci = x.shape[1]
        agg = _scatter_sum(x, src_m, dst_m, n, ci)
        x = _gin_linear(x, agg, params['gin_W'][i], params['gin_b'][i])
        co = x.shape[1]
        aggs = _scatter_sum(x, src_m, dst_m, n, co)
        score = _score(aggs, x, params['p_Wrel'][i], params['p_brel'][i],
                       params['p_Wroot'][i])
        kk = int(math.ceil(float(np.float32(_RATIO) * np.float32(ng))))
        perm = _topk_perm(score, _B, ng, kk)
        x = _bn_scale(x[perm], score[perm], params['bn_g'][i],
                      params['bn_b'][i])
        node_map = jnp.full((n,), -1).at[perm].set(jnp.arange(_B * kk))
        ns, nd = node_map[src], node_map[dst]
        valid = valid & (ns >= 0) & (nd >= 0)
        src = jnp.where(valid, ns, 0)
        dst = jnp.where(valid, nd, 0)
        ng = kk
    n = x.shape[0]
    src_m = jnp.where(valid, src, n)
    dst_m = jnp.where(valid, dst, n)
    aggs = _scatter_sum(x, src_m, dst_m, n, x.shape[1])
    score = _score(aggs, x, params['gp_Wrel'], params['gp_brel'],
                   params['gp_Wroot'])
    perm = _topk_perm(score, _B, ng, ng)
    z = _bn_scale(x[perm], score[perm], params['bn_gf'], params['bn_bf'])
    mu = _linear(z, params['mu_W'], params['mu_b'])
    logvar = _linear(z, params['lv_W'], params['lv_b'])
    return mu, logvar


def kernel(x, params, edge_index, batch, ptr):
    return _forward(x.astype(jnp.float32), params,
                    edge_index[0], edge_index[1])
